# baseline (speedup 1.0000x reference)
import functools

import jax
import jax.numpy as jnp
from jax import lax
from jax.experimental import pallas as pl

B = 32
L = 12
N = 883
C = 3
NPAD = 896
MBLK = 128
NBLK = NPAD // MBLK
HID = 128
EMBED = 32
NODE_DIM = 40
TOPK = 20
TOD = 288
DOW = 7
DOWP = 8
SEQ_OUT = 12
F = L * C
FPAD = 40

_HI = jax.lax.Precision.HIGHEST


def _dot(a, b, prec=_HI):
    return jax.lax.dot_general(a, b, (((1,), (0,)), ((), ())),
                               precision=prec, preferred_element_type=jnp.float32)


def _dot_t(a, b, prec=_HI):
    return jax.lax.dot_general(a, b, (((1,), (1,)), ((), ())),
                               precision=prec, preferred_element_type=jnp.float32)


def _count_cols(mask, ones_col):
    return jax.lax.dot_general(mask.astype(jnp.bfloat16), ones_col,
                               (((1,), (0,)), ((), ())),
                               preferred_element_type=jnp.float32)


def _kth_threshold(adj, k):
    m = adj.shape[0]
    f32, i32 = jnp.float32, jnp.int32
    bits = jax.lax.bitcast_convert_type(adj, i32)
    qa = jax.lax.bitcast_convert_type(
        jnp.bitwise_and(bits, jnp.int32(-65536)), f32)
    qb = qa.astype(jnp.bfloat16)
    kf = float(k)
    ones_col = jnp.ones((adj.shape[1], 1), jnp.bfloat16)

    def grid_val(g):
        return jax.lax.bitcast_convert_type(g << 16, f32)

    def bis_body(_, carry):
        lo, hi, c_hi = carry
        mid = (lo + hi) >> 1
        t = grid_val(mid).astype(jnp.bfloat16)
        c = _count_cols(qb >= t, ones_col)
        take = c >= kf
        return (jnp.where(take, mid, lo), jnp.where(take, hi, mid),
                jnp.where(take, c_hi, c))

    lo0 = jnp.zeros((m, 1), i32)
    hi0 = jnp.full((m, 1), 0x3F81, i32)
    c_hi0 = jnp.zeros((m, 1), f32)
    lo, _, above = lax.fori_loop(0, 14, bis_body, (lo0, hi0, c_hi0))
    qthr = grid_val(lo)
    work = jnp.where(qa == qthr, adj, -1.0)
    candmax = jnp.max(work, axis=1, keepdims=True)
    krem = kf - above
    thr0 = jnp.where(krem <= 1.0, candmax, 2.0)
    kr0 = jnp.where(krem <= 1.0, 0.0, krem)

    def w_cond(carry):
        _, kr, _ = carry
        return jnp.any(kr > 0.0)

    def w_body(carry):
        work, kr, thr = carry
        cur = jnp.max(work, axis=1, keepdims=True)
        eq = work == cur
        cnt = _count_cols(eq, ones_col)
        active = kr > 0.0
        thr = jnp.where(active, cur, thr)
        kr = kr - jnp.where(active, cnt, 0.0)
        work = jnp.where(eq, -1.0, work)
        return work, kr, thr

    _, _, thr = lax.while_loop(w_cond, w_body, (work, kr0, thr0))
    return thr


def _gather_dot(oh, tab):
    hi = tab.astype(jnp.bfloat16).astype(jnp.float32)
    lo = tab - hi
    d = jax.lax.Precision.DEFAULT
    return _dot(oh, hi, d) + _dot(oh, lo, d)


def _prep_body(x_ref, tid_ref, diw_ref, node_ref, tidtab_ref, diwtab_ref,
               wts_ref, bts_ref, w1a_ref, b1a_ref, w1b_ref, b1b_ref,
               w1c_ref, b1c_ref, emb1_ref, h_ref, nv_ref):
    x = x_ref[0]
    ts = _dot(x, wts_ref[...]) + bts_ref[...]
    tid = tid_ref[0]
    diw = diw_ref[0]
    iota_t = lax.broadcasted_iota(jnp.int32, (NPAD, TOD), 1)
    oh_t = (iota_t == tid).astype(jnp.float32)
    emb_t = _gather_dot(oh_t, tidtab_ref[...])
    iota_d = lax.broadcasted_iota(jnp.int32, (NPAD, DOWP), 1)
    oh_d = (iota_d == diw).astype(jnp.float32)
    emb_d = _gather_dot(oh_d, diwtab_ref[...])
    ge = jnp.concatenate([ts, node_ref[...], emb_t, emb_d], axis=1)
    h_ref[0] = ge
    h1 = jax.nn.relu(_dot(ge, w1a_ref[...]) + b1a_ref[...])
    h2 = jax.nn.relu(_dot(h1, w1b_ref[...]) + b1b_ref[...])
    f1 = _dot(h2, w1c_ref[...]) + b1c_ref[...]
    nv_ref[0] = jnp.tanh(emb1_ref[...] * f1)


def _static_body(ed_ref, eu_ref, out_ref):
    r = jax.nn.relu(_dot_t(ed_ref[...], eu_ref[...]))
    col = lax.broadcasted_iota(jnp.int32, (MBLK, NPAD), 1)
    valid = col < N
    r = jnp.where(valid, r, -jnp.inf)
    mx = jnp.max(r, axis=1, keepdims=True)
    e = jnp.where(valid, jnp.exp(r - mx), 0.0)
    sg = e / jnp.sum(e, axis=1, keepdims=True)
    thr = _kth_threshold(sg, TOPK)
    gt = sg > thr
    n_gt = _count_cols(gt, jnp.ones((NPAD, 1), jnp.bfloat16))
    need = float(TOPK) - n_gt
    tie = (sg == thr).astype(jnp.float32)
    ia = lax.broadcasted_iota(jnp.int32, (NPAD, NPAD), 0)
    ib = lax.broadcasted_iota(jnp.int32, (NPAD, NPAD), 1)
    tri = (ia < ib).astype(jnp.float32)
    prefix = jax.lax.dot_general(tie, tri, (((1,), (0,)), ((), ())),
                                 preferred_element_type=jnp.float32)
    keep = gt | ((tie > 0.0) & (prefix < need))
    out_ref[...] = jnp.where(keep, sg, 0.0)


def _fused_body(nv_ref, h_ref, sg_ref, wf_ref, bf_ref, out_ref):
    i = pl.program_id(1)
    nv = nv_ref[0]
    nvb = nv_ref[0, pl.ds(i * MBLK, MBLK), :]
    a = _dot_t(nvb, nv)
    adj = jax.nn.relu(jnp.tanh(a))
    thr = _kth_threshold(adj, TOPK)
    dyn = jnp.where(adj >= thr, adj, 0.0)
    h = h_ref[0]
    dflt = jax.lax.Precision.DEFAULT
    hd = _dot(dyn, h, dflt)
    hs = _dot(sg_ref[pl.ds(i * MBLK, MBLK), :], h, dflt)
    hres = h_ref[0, pl.ds(i * MBLK, MBLK), :]
    fused = jax.nn.relu(hs + hd + hres)
    out_ref[0] = _dot(fused, wf_ref[...], dflt) + bf_ref[...]


def _pad_rows(arr, rows):
    return jnp.pad(arr, ((0, rows - arr.shape[0]),) + ((0, 0),) * (arr.ndim - 1))


@jax.jit
def kernel(history_data, TID, DIW, node_emb, node_emb_u, node_emb_d, emb1_w,
           emb2_w, Wts, bts, W1a, b1a, W1b, b1b, W1c, b1c, W2a, b2a, W2b,
           b2b, W2c, b2c, W_fore, b_fore):
    f32 = jnp.float32
    x = history_data.transpose(0, 2, 1, 3).reshape(B, N, F)
    x = jnp.pad(x, ((0, 0), (0, NPAD - N), (0, FPAD - F)))
    tid_idx = (history_data[:, -1, :, 1] * TOD).astype(jnp.int32)
    diw_idx = (history_data[:, -1, :, 2] * DOW).astype(jnp.int32)
    tid_idx = jnp.pad(tid_idx, ((0, 0), (0, NPAD - N)))[..., None]
    diw_idx = jnp.pad(diw_idx, ((0, 0), (0, NPAD - N)))[..., None]

    node_p = _pad_rows(node_emb, NPAD)
    emb1_p = _pad_rows(emb1_w, NPAD)
    eu_p = _pad_rows(node_emb_u, NPAD)
    ed_p = _pad_rows(node_emb_d, NPAD)
    diw_tab = _pad_rows(DIW, DOWP)
    wts_p = _pad_rows(Wts, FPAD)

    h, nv = pl.pallas_call(
        _prep_body,
        grid=(B,),
        in_specs=[
            pl.BlockSpec((1, NPAD, FPAD), lambda b: (b, 0, 0)),
            pl.BlockSpec((1, NPAD, 1), lambda b: (b, 0, 0)),
            pl.BlockSpec((1, NPAD, 1), lambda b: (b, 0, 0)),
            pl.BlockSpec((NPAD, EMBED), lambda b: (0, 0)),
            pl.BlockSpec((TOD, EMBED), lambda b: (0, 0)),
            pl.BlockSpec((DOWP, EMBED), lambda b: (0, 0)),
            pl.BlockSpec((FPAD, EMBED), lambda b: (0, 0)),
            pl.BlockSpec((1, EMBED), lambda b: (0, 0)),
            pl.BlockSpec((HID, 64), lambda b: (0, 0)),
            pl.BlockSpec((1, 64), lambda b: (0, 0)),
            pl.BlockSpec((64, 64), lambda b: (0, 0)),
            pl.BlockSpec((1, 64), lambda b: (0, 0)),
            pl.BlockSpec((64, NODE_DIM), lambda b: (0, 0)),
            pl.BlockSpec((1, NODE_DIM), lambda b: (0, 0)),
            pl.BlockSpec((NPAD, NODE_DIM), lambda b: (0, 0)),
        ],
        out_specs=[
            pl.BlockSpec((1, NPAD, HID), lambda b: (b, 0, 0)),
            pl.BlockSpec((1, NPAD, NODE_DIM), lambda b: (b, 0, 0)),
        ],
        out_shape=[
            jax.ShapeDtypeStruct((B, NPAD, HID), f32),
            jax.ShapeDtypeStruct((B, NPAD, NODE_DIM), f32),
        ],
    )(x, tid_idx, diw_idx, node_p, TID, diw_tab, wts_p, bts[None],
      W1a, b1a[None], W1b, b1b[None], W1c, b1c[None], emb1_p)

    static_graph = pl.pallas_call(
        _static_body,
        grid=(NBLK,),
        in_specs=[
            pl.BlockSpec((MBLK, EMBED), lambda i: (i, 0)),
            pl.BlockSpec((NPAD, EMBED), lambda i: (0, 0)),
        ],
        out_specs=pl.BlockSpec((MBLK, NPAD), lambda i: (i, 0)),
        out_shape=jax.ShapeDtypeStruct((NPAD, NPAD), f32),
    )(ed_p, eu_p)

    out = pl.pallas_call(
        _fused_body,
        grid=(B, NBLK),
        in_specs=[
            pl.BlockSpec((1, NPAD, NODE_DIM), lambda b, i: (b, 0, 0)),
            pl.BlockSpec((1, NPAD, HID), lambda b, i: (b, 0, 0)),
            pl.BlockSpec((NPAD, NPAD), lambda b, i: (0, 0)),
            pl.BlockSpec((HID, SEQ_OUT), lambda b, i: (0, 0)),
            pl.BlockSpec((1, SEQ_OUT), lambda b, i: (0, 0)),
        ],
        out_specs=pl.BlockSpec((1, MBLK, SEQ_OUT), lambda b, i: (b, i, 0)),
        out_shape=jax.ShapeDtypeStruct((B, NPAD, SEQ_OUT), f32),
    )(nv, h, static_graph, W_fore, b_fore[None])

    return out[:, :N, :]


# speedup vs baseline: 1.2642x; 1.2642x over previous
import functools

import jax
import jax.numpy as jnp
from jax import lax
from jax.experimental import pallas as pl

B = 32
L = 12
N = 883
C = 3
NPAD = 896
MBLK = 128
NBLK = NPAD // MBLK
HID = 128
EMBED = 32
NODE_DIM = 40
TOPK = 20
TOD = 288
DOW = 7
DOWP = 8
SEQ_OUT = 12
F = L * C
FPAD = 40

_HI = jax.lax.Precision.HIGHEST


def _dot(a, b, prec=_HI):
    return jax.lax.dot_general(a, b, (((1,), (0,)), ((), ())),
                               precision=prec, preferred_element_type=jnp.float32)


def _dot_t(a, b, prec=_HI):
    return jax.lax.dot_general(a, b, (((1,), (1,)), ((), ())),
                               precision=prec, preferred_element_type=jnp.float32)


def _count_cols(mask, ones_col):
    return jax.lax.dot_general(mask.astype(jnp.bfloat16), ones_col,
                               (((1,), (0,)), ((), ())),
                               preferred_element_type=jnp.float32)


def _kth_threshold(adj, k):
    m = adj.shape[0]
    f32, i32 = jnp.float32, jnp.int32
    bits = jax.lax.bitcast_convert_type(adj, i32)
    qa = jax.lax.bitcast_convert_type(
        jnp.bitwise_and(bits, jnp.int32(-65536)), f32)
    qb = qa.astype(jnp.bfloat16)
    kf = float(k)

    def grid_val(g):
        return jax.lax.bitcast_convert_type(g << 16, f32)

    def bis_body(_, carry):
        lo, hi, c_hi = carry
        mid = (lo + hi) >> 1
        t = grid_val(mid).astype(jnp.bfloat16)
        c = jnp.sum((qb >= t).astype(f32), axis=1, keepdims=True)
        take = c >= kf
        return (jnp.where(take, mid, lo), jnp.where(take, hi, mid),
                jnp.where(take, c_hi, c))

    lo0 = jnp.zeros((m, 1), i32)
    hi0 = jnp.full((m, 1), 0x3F81, i32)
    c_hi0 = jnp.zeros((m, 1), f32)
    lo, _, above = lax.fori_loop(0, 14, bis_body, (lo0, hi0, c_hi0))
    qthr = grid_val(lo)
    work = jnp.where(qa == qthr, adj, -1.0)
    candmax = jnp.max(work, axis=1, keepdims=True)
    krem = kf - above
    thr0 = jnp.where(krem <= 1.0, candmax, 2.0)
    kr0 = jnp.where(krem <= 1.0, 0.0, krem)

    def w_cond(carry):
        _, kr, _ = carry
        return jnp.any(kr > 0.0)

    def w_body(carry):
        work, kr, thr = carry
        cur = jnp.max(work, axis=1, keepdims=True)
        eq = work == cur
        cnt = jnp.sum(eq.astype(f32), axis=1, keepdims=True)
        active = kr > 0.0
        thr = jnp.where(active, cur, thr)
        kr = kr - jnp.where(active, cnt, 0.0)
        work = jnp.where(eq, -1.0, work)
        return work, kr, thr

    _, _, thr = lax.while_loop(w_cond, w_body, (work, kr0, thr0))
    return thr


def _kth_threshold_cols(adj, k):
    w, m = adj.shape
    f32, i32 = jnp.float32, jnp.int32
    g = w // MBLK
    bits = jax.lax.bitcast_convert_type(adj, i32)
    qa = jax.lax.bitcast_convert_type(
        jnp.bitwise_and(bits, jnp.int32(-65536)), f32)
    qb = qa.astype(jnp.bfloat16)
    kf = float(k)

    def grid_val(gv):
        return jax.lax.bitcast_convert_type(gv << 16, f32)

    def colcount(mask):
        s7 = jnp.sum(mask.astype(jnp.bfloat16).reshape(g, MBLK, m), axis=0)
        return jnp.sum(s7.astype(f32), axis=0, keepdims=True)

    def bis_body(_, carry):
        lo, hi, c_hi = carry
        mid = (lo + hi) >> 1
        t = grid_val(mid).astype(jnp.bfloat16)
        c = colcount(qb >= t)
        take = c >= kf
        return (jnp.where(take, mid, lo), jnp.where(take, hi, mid),
                jnp.where(take, c_hi, c))

    lo0 = jnp.zeros((1, m), i32)
    hi0 = jnp.full((1, m), 0x3F81, i32)
    c_hi0 = jnp.zeros((1, m), f32)
    lo, _, above = lax.fori_loop(0, 14, bis_body, (lo0, hi0, c_hi0))
    qthr = grid_val(lo)
    work = jnp.where(qa == qthr, adj, -1.0)
    candmax = jnp.max(work, axis=0, keepdims=True)
    krem = kf - above
    thr0 = jnp.where(krem <= 1.0, candmax, 2.0)
    kr0 = jnp.where(krem <= 1.0, 0.0, krem)

    def w_cond(carry):
        _, kr, _ = carry
        return jnp.any(kr > 0.0)

    def w_body(carry):
        work, kr, thr = carry
        cur = jnp.max(work, axis=0, keepdims=True)
        eq = work == cur
        cnt = colcount(eq)
        active = kr > 0.0
        thr = jnp.where(active, cur, thr)
        kr = kr - jnp.where(active, cnt, 0.0)
        work = jnp.where(eq, -1.0, work)
        return work, kr, thr

    _, _, thr = lax.while_loop(w_cond, w_body, (work, kr0, thr0))
    return thr


def _gather_dot(oh, tab):
    hi = tab.astype(jnp.bfloat16).astype(jnp.float32)
    lo = tab - hi
    d = jax.lax.Precision.DEFAULT
    return _dot(oh, hi, d) + _dot(oh, lo, d)


def _prep_body(x_ref, tid_ref, diw_ref, node_ref, tidtab_ref, diwtab_ref,
               wts_ref, bts_ref, w1a_ref, b1a_ref, w1b_ref, b1b_ref,
               w1c_ref, b1c_ref, emb1_ref, h_ref, nv_ref):
    x = x_ref[0]
    ts = _dot(x, wts_ref[...]) + bts_ref[...]
    tid = tid_ref[0]
    diw = diw_ref[0]
    iota_t = lax.broadcasted_iota(jnp.int32, (NPAD, TOD), 1)
    oh_t = (iota_t == tid).astype(jnp.float32)
    emb_t = _gather_dot(oh_t, tidtab_ref[...])
    iota_d = lax.broadcasted_iota(jnp.int32, (NPAD, DOWP), 1)
    oh_d = (iota_d == diw).astype(jnp.float32)
    emb_d = _gather_dot(oh_d, diwtab_ref[...])
    ge = jnp.concatenate([ts, node_ref[...], emb_t, emb_d], axis=1)
    h_ref[0] = ge
    h1 = jax.nn.relu(_dot(ge, w1a_ref[...]) + b1a_ref[...])
    h2 = jax.nn.relu(_dot(h1, w1b_ref[...]) + b1b_ref[...])
    f1 = _dot(h2, w1c_ref[...]) + b1c_ref[...]
    nv_ref[0] = jnp.tanh(emb1_ref[...] * f1)


def _static_body(ed_ref, eu_ref, out_ref):
    r = jax.nn.relu(_dot_t(ed_ref[...], eu_ref[...]))
    col = lax.broadcasted_iota(jnp.int32, (MBLK, NPAD), 1)
    valid = col < N
    r = jnp.where(valid, r, -jnp.inf)
    mx = jnp.max(r, axis=1, keepdims=True)
    e = jnp.where(valid, jnp.exp(r - mx), 0.0)
    sg = e / jnp.sum(e, axis=1, keepdims=True)
    thr = _kth_threshold(sg, TOPK)
    gt = sg > thr
    n_gt = _count_cols(gt, jnp.ones((NPAD, 1), jnp.bfloat16))
    need = float(TOPK) - n_gt
    tie = (sg == thr).astype(jnp.float32)
    ia = lax.broadcasted_iota(jnp.int32, (NPAD, NPAD), 0)
    ib = lax.broadcasted_iota(jnp.int32, (NPAD, NPAD), 1)
    tri = (ia < ib).astype(jnp.float32)
    prefix = jax.lax.dot_general(tie, tri, (((1,), (0,)), ((), ())),
                                 preferred_element_type=jnp.float32)
    keep = gt | ((tie > 0.0) & (prefix < need))
    out_ref[...] = jnp.where(keep, sg, 0.0)


def _fused_body(nv_ref, h_ref, sg_ref, wf_ref, bf_ref, out_ref):
    i = pl.program_id(1)
    nv = nv_ref[0]
    nvb = nv_ref[0, pl.ds(i * MBLK, MBLK), :]
    a_t = _dot_t(nv, nvb)
    adj_t = jax.nn.relu(jnp.tanh(a_t))
    thr = _kth_threshold_cols(adj_t, TOPK)
    dyn_t = jnp.where(adj_t >= thr, adj_t, 0.0)
    h = h_ref[0]
    dflt = jax.lax.Precision.DEFAULT
    hd = jax.lax.dot_general(dyn_t, h, (((0,), (0,)), ((), ())),
                             precision=dflt,
                             preferred_element_type=jnp.float32)
    hs = _dot(sg_ref[pl.ds(i * MBLK, MBLK), :], h, dflt)
    hres = h_ref[0, pl.ds(i * MBLK, MBLK), :]
    fused = jax.nn.relu(hs + hd + hres)
    out_ref[0] = _dot(fused, wf_ref[...], dflt) + bf_ref[...]


def _pad_rows(arr, rows):
    return jnp.pad(arr, ((0, rows - arr.shape[0]),) + ((0, 0),) * (arr.ndim - 1))


@jax.jit
def kernel(history_data, TID, DIW, node_emb, node_emb_u, node_emb_d, emb1_w,
           emb2_w, Wts, bts, W1a, b1a, W1b, b1b, W1c, b1c, W2a, b2a, W2b,
           b2b, W2c, b2c, W_fore, b_fore):
    f32 = jnp.float32
    x = history_data.transpose(0, 2, 1, 3).reshape(B, N, F)
    x = jnp.pad(x, ((0, 0), (0, NPAD - N), (0, FPAD - F)))
    tid_idx = (history_data[:, -1, :, 1] * TOD).astype(jnp.int32)
    diw_idx = (history_data[:, -1, :, 2] * DOW).astype(jnp.int32)
    tid_idx = jnp.pad(tid_idx, ((0, 0), (0, NPAD - N)))[..., None]
    diw_idx = jnp.pad(diw_idx, ((0, 0), (0, NPAD - N)))[..., None]

    node_p = _pad_rows(node_emb, NPAD)
    emb1_p = _pad_rows(emb1_w, NPAD)
    eu_p = _pad_rows(node_emb_u, NPAD)
    ed_p = _pad_rows(node_emb_d, NPAD)
    diw_tab = _pad_rows(DIW, DOWP)
    wts_p = _pad_rows(Wts, FPAD)

    h, nv = pl.pallas_call(
        _prep_body,
        grid=(B,),
        in_specs=[
            pl.BlockSpec((1, NPAD, FPAD), lambda b: (b, 0, 0)),
            pl.BlockSpec((1, NPAD, 1), lambda b: (b, 0, 0)),
            pl.BlockSpec((1, NPAD, 1), lambda b: (b, 0, 0)),
            pl.BlockSpec((NPAD, EMBED), lambda b: (0, 0)),
            pl.BlockSpec((TOD, EMBED), lambda b: (0, 0)),
            pl.BlockSpec((DOWP, EMBED), lambda b: (0, 0)),
            pl.BlockSpec((FPAD, EMBED), lambda b: (0, 0)),
            pl.BlockSpec((1, EMBED), lambda b: (0, 0)),
            pl.BlockSpec((HID, 64), lambda b: (0, 0)),
            pl.BlockSpec((1, 64), lambda b: (0, 0)),
            pl.BlockSpec((64, 64), lambda b: (0, 0)),
            pl.BlockSpec((1, 64), lambda b: (0, 0)),
            pl.BlockSpec((64, NODE_DIM), lambda b: (0, 0)),
            pl.BlockSpec((1, NODE_DIM), lambda b: (0, 0)),
            pl.BlockSpec((NPAD, NODE_DIM), lambda b: (0, 0)),
        ],
        out_specs=[
            pl.BlockSpec((1, NPAD, HID), lambda b: (b, 0, 0)),
            pl.BlockSpec((1, NPAD, NODE_DIM), lambda b: (b, 0, 0)),
        ],
        out_shape=[
            jax.ShapeDtypeStruct((B, NPAD, HID), f32),
            jax.ShapeDtypeStruct((B, NPAD, NODE_DIM), f32),
        ],
    )(x, tid_idx, diw_idx, node_p, TID, diw_tab, wts_p, bts[None],
      W1a, b1a[None], W1b, b1b[None], W1c, b1c[None], emb1_p)

    static_graph = pl.pallas_call(
        _static_body,
        grid=(NBLK,),
        in_specs=[
            pl.BlockSpec((MBLK, EMBED), lambda i: (i, 0)),
            pl.BlockSpec((NPAD, EMBED), lambda i: (0, 0)),
        ],
        out_specs=pl.BlockSpec((MBLK, NPAD), lambda i: (i, 0)),
        out_shape=jax.ShapeDtypeStruct((NPAD, NPAD), f32),
    )(ed_p, eu_p)

    out = pl.pallas_call(
        _fused_body,
        grid=(B, NBLK),
        in_specs=[
            pl.BlockSpec((1, NPAD, NODE_DIM), lambda b, i: (b, 0, 0)),
            pl.BlockSpec((1, NPAD, HID), lambda b, i: (b, 0, 0)),
            pl.BlockSpec((NPAD, NPAD), lambda b, i: (0, 0)),
            pl.BlockSpec((HID, SEQ_OUT), lambda b, i: (0, 0)),
            pl.BlockSpec((1, SEQ_OUT), lambda b, i: (0, 0)),
        ],
        out_specs=pl.BlockSpec((1, MBLK, SEQ_OUT), lambda b, i: (b, i, 0)),
        out_shape=jax.ShapeDtypeStruct((B, NPAD, SEQ_OUT), f32),
    )(nv, h, static_graph, W_fore, b_fore[None])

    return out[:, :N, :]


# speedup vs baseline: 1.8687x; 1.4781x over previous
import functools

import jax
import jax.numpy as jnp
from jax import lax
from jax.experimental import pallas as pl

B = 32
L = 12
N = 883
C = 3
NPAD = 896
MBLK = 128
NBLK = NPAD // MBLK
HID = 128
EMBED = 32
NODE_DIM = 40
TOPK = 20
TOD = 288
DOW = 7
DOWP = 8
SEQ_OUT = 12
F = L * C
FPAD = 40

_HI = jax.lax.Precision.HIGHEST


def _dot(a, b, prec=_HI):
    return jax.lax.dot_general(a, b, (((1,), (0,)), ((), ())),
                               precision=prec, preferred_element_type=jnp.float32)


def _dot_t(a, b, prec=_HI):
    return jax.lax.dot_general(a, b, (((1,), (1,)), ((), ())),
                               precision=prec, preferred_element_type=jnp.float32)


def _count_cols(mask, ones_col):
    return jax.lax.dot_general(mask.astype(jnp.bfloat16), ones_col,
                               (((1,), (0,)), ((), ())),
                               preferred_element_type=jnp.float32)


def _kth_threshold(adj, k):
    m = adj.shape[0]
    f32, i32 = jnp.float32, jnp.int32
    bits = jax.lax.bitcast_convert_type(adj, i32)
    qa = jax.lax.bitcast_convert_type(
        jnp.bitwise_and(bits, jnp.int32(-65536)), f32)
    qb = qa.astype(jnp.bfloat16)
    kf = float(k)

    def grid_val(g):
        return jax.lax.bitcast_convert_type(g << 16, f32)

    lo = jnp.zeros((m, 1), i32)
    hi = jnp.full((m, 1), 0x3F81, i32)
    above = jnp.zeros((m, 1), f32)
    for _ in range(14):
        mid = (lo + hi) >> 1
        t = grid_val(mid).astype(jnp.bfloat16)
        c = jnp.sum((qb >= t).astype(f32), axis=1, keepdims=True)
        take = c >= kf
        lo = jnp.where(take, mid, lo)
        hi = jnp.where(take, hi, mid)
        above = jnp.where(take, above, c)
    qthr = grid_val(lo)
    work = jnp.where(qa == qthr, adj, -1.0)
    krem = kf - above

    def _fix_step(carry):
        work, kr, thr = carry
        cur = jnp.max(work, axis=1, keepdims=True)
        eq = work == cur
        cnt = jnp.sum(eq.astype(f32), axis=1, keepdims=True)
        active = kr > 0.0
        thr = jnp.where(active, cur, thr)
        kr = kr - jnp.where(active, cnt, 0.0)
        work = jnp.where(eq, -1.0, work)
        return work, kr, thr

    carry = (work, krem, jnp.full((m, 1), 2.0, f32))
    carry = _fix_step(carry)
    carry = _fix_step(carry)
    _, _, thr = lax.while_loop(lambda cy: jnp.any(cy[1] > 0.0), _fix_step,
                               carry)
    return thr


def _gather_dot(oh, tab):
    hi = tab.astype(jnp.bfloat16).astype(jnp.float32)
    lo = tab - hi
    d = jax.lax.Precision.DEFAULT
    return _dot(oh, hi, d) + _dot(oh, lo, d)


def _prep_body(x_ref, tid_ref, diw_ref, node_ref, tidtab_ref, diwtab_ref,
               wts_ref, bts_ref, w1a_ref, b1a_ref, w1b_ref, b1b_ref,
               w1c_ref, b1c_ref, emb1_ref, h_ref, nv_ref):
    x = x_ref[0]
    ts = _dot(x, wts_ref[...]) + bts_ref[...]
    tid = tid_ref[0]
    diw = diw_ref[0]
    iota_t = lax.broadcasted_iota(jnp.int32, (NPAD, TOD), 1)
    oh_t = (iota_t == tid).astype(jnp.float32)
    emb_t = _gather_dot(oh_t, tidtab_ref[...])
    iota_d = lax.broadcasted_iota(jnp.int32, (NPAD, DOWP), 1)
    oh_d = (iota_d == diw).astype(jnp.float32)
    emb_d = _gather_dot(oh_d, diwtab_ref[...])
    ge = jnp.concatenate([ts, node_ref[...], emb_t, emb_d], axis=1)
    h_ref[0] = ge
    h1 = jax.nn.relu(_dot(ge, w1a_ref[...]) + b1a_ref[...])
    h2 = jax.nn.relu(_dot(h1, w1b_ref[...]) + b1b_ref[...])
    f1 = _dot(h2, w1c_ref[...]) + b1c_ref[...]
    nv_ref[0] = jnp.tanh(emb1_ref[...] * f1)


def _static_body(ed_ref, eu_ref, out_ref):
    r = jax.nn.relu(_dot_t(ed_ref[...], eu_ref[...]))
    col = lax.broadcasted_iota(jnp.int32, (MBLK, NPAD), 1)
    valid = col < N
    r = jnp.where(valid, r, -jnp.inf)
    mx = jnp.max(r, axis=1, keepdims=True)
    e = jnp.where(valid, jnp.exp(r - mx), 0.0)
    sg = e / jnp.sum(e, axis=1, keepdims=True)
    thr = _kth_threshold(sg, TOPK)
    gt = sg > thr
    n_gt = _count_cols(gt, jnp.ones((NPAD, 1), jnp.bfloat16))
    need = float(TOPK) - n_gt
    tie = (sg == thr).astype(jnp.float32)
    ia = lax.broadcasted_iota(jnp.int32, (NPAD, NPAD), 0)
    ib = lax.broadcasted_iota(jnp.int32, (NPAD, NPAD), 1)
    tri = (ia < ib).astype(jnp.float32)
    prefix = jax.lax.dot_general(tie, tri, (((1,), (0,)), ((), ())),
                                 preferred_element_type=jnp.float32)
    keep = gt | ((tie > 0.0) & (prefix < need))
    out_ref[...] = jnp.where(keep, sg, 0.0)


def _fused_body(nv_ref, h_ref, sg_ref, wf_ref, bf_ref, out_ref):
    i = pl.program_id(1)
    nv = nv_ref[0]
    nvb = nv_ref[0, pl.ds(i * MBLK, MBLK), :]
    a = _dot_t(nvb, nv)
    adj = jax.nn.relu(jnp.tanh(a))
    thr = _kth_threshold(adj, TOPK)
    dyn = jnp.where(adj >= thr, adj, 0.0)
    h = h_ref[0]
    dflt = jax.lax.Precision.DEFAULT
    hd = _dot(dyn, h, dflt)
    hs = _dot(sg_ref[pl.ds(i * MBLK, MBLK), :], h, dflt)
    hres = h_ref[0, pl.ds(i * MBLK, MBLK), :]
    fused = jax.nn.relu(hs + hd + hres)
    out_ref[0] = _dot(fused, wf_ref[...], dflt) + bf_ref[...]


def _pad_rows(arr, rows):
    return jnp.pad(arr, ((0, rows - arr.shape[0]),) + ((0, 0),) * (arr.ndim - 1))


@jax.jit
def kernel(history_data, TID, DIW, node_emb, node_emb_u, node_emb_d, emb1_w,
           emb2_w, Wts, bts, W1a, b1a, W1b, b1b, W1c, b1c, W2a, b2a, W2b,
           b2b, W2c, b2c, W_fore, b_fore):
    f32 = jnp.float32
    x = history_data.transpose(0, 2, 1, 3).reshape(B, N, F)
    x = jnp.pad(x, ((0, 0), (0, NPAD - N), (0, FPAD - F)))
    tid_idx = (history_data[:, -1, :, 1] * TOD).astype(jnp.int32)
    diw_idx = (history_data[:, -1, :, 2] * DOW).astype(jnp.int32)
    tid_idx = jnp.pad(tid_idx, ((0, 0), (0, NPAD - N)))[..., None]
    diw_idx = jnp.pad(diw_idx, ((0, 0), (0, NPAD - N)))[..., None]

    node_p = _pad_rows(node_emb, NPAD)
    emb1_p = _pad_rows(emb1_w, NPAD)
    eu_p = _pad_rows(node_emb_u, NPAD)
    ed_p = _pad_rows(node_emb_d, NPAD)
    diw_tab = _pad_rows(DIW, DOWP)
    wts_p = _pad_rows(Wts, FPAD)

    h, nv = pl.pallas_call(
        _prep_body,
        grid=(B,),
        in_specs=[
            pl.BlockSpec((1, NPAD, FPAD), lambda b: (b, 0, 0)),
            pl.BlockSpec((1, NPAD, 1), lambda b: (b, 0, 0)),
            pl.BlockSpec((1, NPAD, 1), lambda b: (b, 0, 0)),
            pl.BlockSpec((NPAD, EMBED), lambda b: (0, 0)),
            pl.BlockSpec((TOD, EMBED), lambda b: (0, 0)),
            pl.BlockSpec((DOWP, EMBED), lambda b: (0, 0)),
            pl.BlockSpec((FPAD, EMBED), lambda b: (0, 0)),
            pl.BlockSpec((1, EMBED), lambda b: (0, 0)),
            pl.BlockSpec((HID, 64), lambda b: (0, 0)),
            pl.BlockSpec((1, 64), lambda b: (0, 0)),
            pl.BlockSpec((64, 64), lambda b: (0, 0)),
            pl.BlockSpec((1, 64), lambda b: (0, 0)),
            pl.BlockSpec((64, NODE_DIM), lambda b: (0, 0)),
            pl.BlockSpec((1, NODE_DIM), lambda b: (0, 0)),
            pl.BlockSpec((NPAD, NODE_DIM), lambda b: (0, 0)),
        ],
        out_specs=[
            pl.BlockSpec((1, NPAD, HID), lambda b: (b, 0, 0)),
            pl.BlockSpec((1, NPAD, NODE_DIM), lambda b: (b, 0, 0)),
        ],
        out_shape=[
            jax.ShapeDtypeStruct((B, NPAD, HID), f32),
            jax.ShapeDtypeStruct((B, NPAD, NODE_DIM), f32),
        ],
    )(x, tid_idx, diw_idx, node_p, TID, diw_tab, wts_p, bts[None],
      W1a, b1a[None], W1b, b1b[None], W1c, b1c[None], emb1_p)

    static_graph = pl.pallas_call(
        _static_body,
        grid=(NBLK,),
        in_specs=[
            pl.BlockSpec((MBLK, EMBED), lambda i: (i, 0)),
            pl.BlockSpec((NPAD, EMBED), lambda i: (0, 0)),
        ],
        out_specs=pl.BlockSpec((MBLK, NPAD), lambda i: (i, 0)),
        out_shape=jax.ShapeDtypeStruct((NPAD, NPAD), f32),
    )(ed_p, eu_p)

    out = pl.pallas_call(
        _fused_body,
        grid=(B, NBLK),
        in_specs=[
            pl.BlockSpec((1, NPAD, NODE_DIM), lambda b, i: (b, 0, 0)),
            pl.BlockSpec((1, NPAD, HID), lambda b, i: (b, 0, 0)),
            pl.BlockSpec((NPAD, NPAD), lambda b, i: (0, 0)),
            pl.BlockSpec((HID, SEQ_OUT), lambda b, i: (0, 0)),
            pl.BlockSpec((1, SEQ_OUT), lambda b, i: (0, 0)),
        ],
        out_specs=pl.BlockSpec((1, MBLK, SEQ_OUT), lambda b, i: (b, i, 0)),
        out_shape=jax.ShapeDtypeStruct((B, NPAD, SEQ_OUT), f32),
    )(nv, h, static_graph, W_fore, b_fore[None])

    return out[:, :N, :]


# speedup vs baseline: 2.4122x; 1.2909x over previous
import functools

import jax
import jax.numpy as jnp
from jax import lax
from jax.experimental import pallas as pl

B = 32
L = 12
N = 883
C = 3
NPAD = 896
MBLK = 448
NBLK = NPAD // MBLK
HID = 128
EMBED = 32
NODE_DIM = 40
TOPK = 20
TOD = 288
DOW = 7
DOWP = 8
SEQ_OUT = 12
F = L * C
FPAD = 40

_HI = jax.lax.Precision.HIGHEST


def _dot(a, b, prec=_HI):
    return jax.lax.dot_general(a, b, (((1,), (0,)), ((), ())),
                               precision=prec, preferred_element_type=jnp.float32)


def _dot_t(a, b, prec=_HI):
    return jax.lax.dot_general(a, b, (((1,), (1,)), ((), ())),
                               precision=prec, preferred_element_type=jnp.float32)


def _split_hl(x):
    h = x.astype(jnp.bfloat16).astype(jnp.float32)
    return h, x - h


def _dot3(a, b):
    d = jax.lax.Precision.DEFAULT
    ah, al = _split_hl(a)
    bh, bl = _split_hl(b)
    return _dot(ah, bh, d) + (_dot(ah, bl, d) + _dot(al, bh, d))


def _count_cols(mask, ones_col):
    return jax.lax.dot_general(mask.astype(jnp.bfloat16), ones_col,
                               (((1,), (0,)), ((), ())),
                               preferred_element_type=jnp.float32)


def _kth_threshold(adj, k):
    m = adj.shape[0]
    f32, i32 = jnp.float32, jnp.int32
    bits = jax.lax.bitcast_convert_type(adj, i32)
    qa = jax.lax.bitcast_convert_type(
        jnp.bitwise_and(bits, jnp.int32(-65536)), f32)
    qb = qa.astype(jnp.bfloat16)
    kf = float(k)

    def grid_val(g):
        return jax.lax.bitcast_convert_type(g << 16, f32)

    lo = jnp.zeros((m, 1), i32)
    hi = jnp.full((m, 1), 0x3F81, i32)
    above = jnp.zeros((m, 1), f32)
    for _ in range(14):
        mid = (lo + hi) >> 1
        t = grid_val(mid).astype(jnp.bfloat16)
        c = jnp.sum((qb >= t).astype(f32), axis=1, keepdims=True)
        take = c >= kf
        lo = jnp.where(take, mid, lo)
        hi = jnp.where(take, hi, mid)
        above = jnp.where(take, above, c)
    qthr = grid_val(lo)
    work = jnp.where(qa == qthr, adj, -1.0)
    krem = kf - above

    def _fix_step(carry):
        work, kr, thr = carry
        cur = jnp.max(work, axis=1, keepdims=True)
        eq = work == cur
        cnt = jnp.sum(eq.astype(f32), axis=1, keepdims=True)
        active = kr > 0.0
        thr = jnp.where(active, cur, thr)
        kr = kr - jnp.where(active, cnt, 0.0)
        work = jnp.where(eq, -1.0, work)
        return work, kr, thr

    carry = (work, krem, jnp.full((m, 1), 2.0, f32))
    carry = _fix_step(carry)
    carry = _fix_step(carry)
    _, _, thr = lax.while_loop(lambda cy: jnp.any(cy[1] > 0.0), _fix_step,
                               carry)
    return thr


def _gather_dot(oh, tab):
    hi = tab.astype(jnp.bfloat16).astype(jnp.float32)
    lo = tab - hi
    d = jax.lax.Precision.DEFAULT
    return _dot(oh, hi, d) + _dot(oh, lo, d)


def _prep_body(x_ref, tid_ref, diw_ref, node_ref, tidtab_ref, diwtab_ref,
               wts_ref, bts_ref, w1a_ref, b1a_ref, w1b_ref, b1b_ref,
               w1c_ref, b1c_ref, emb1_ref, h_ref, nv_ref):
    x = x_ref[0]
    ts = _dot3(x, wts_ref[...]) + bts_ref[...]
    tid = tid_ref[0]
    diw = diw_ref[0]
    iota_t = lax.broadcasted_iota(jnp.int32, (NPAD, TOD), 1)
    oh_t = (iota_t == tid).astype(jnp.float32)
    emb_t = _gather_dot(oh_t, tidtab_ref[...])
    iota_d = lax.broadcasted_iota(jnp.int32, (NPAD, DOWP), 1)
    oh_d = (iota_d == diw).astype(jnp.float32)
    emb_d = _gather_dot(oh_d, diwtab_ref[...])
    ge = jnp.concatenate([ts, node_ref[...], emb_t, emb_d], axis=1)
    h_ref[0] = ge
    h1 = jax.nn.relu(_dot3(ge, w1a_ref[...]) + b1a_ref[...])
    h2 = jax.nn.relu(_dot3(h1, w1b_ref[...]) + b1b_ref[...])
    f1 = _dot3(h2, w1c_ref[...]) + b1c_ref[...]
    nv_ref[0] = jnp.tanh(emb1_ref[...] * f1)


def _static_body(ed_ref, eu_ref, out_ref):
    r = jax.nn.relu(_dot_t(ed_ref[...], eu_ref[...]))
    col = lax.broadcasted_iota(jnp.int32, (MBLK, NPAD), 1)
    valid = col < N
    r = jnp.where(valid, r, -jnp.inf)
    mx = jnp.max(r, axis=1, keepdims=True)
    e = jnp.where(valid, jnp.exp(r - mx), 0.0)
    sg = e / jnp.sum(e, axis=1, keepdims=True)
    thr = _kth_threshold(sg, TOPK)
    gt = sg > thr
    n_gt = _count_cols(gt, jnp.ones((NPAD, 1), jnp.bfloat16))
    need = float(TOPK) - n_gt
    tie = (sg == thr).astype(jnp.float32)
    ia = lax.broadcasted_iota(jnp.int32, (NPAD, NPAD), 0)
    ib = lax.broadcasted_iota(jnp.int32, (NPAD, NPAD), 1)
    tri = (ia < ib).astype(jnp.float32)
    prefix = jax.lax.dot_general(tie, tri, (((1,), (0,)), ((), ())),
                                 preferred_element_type=jnp.float32)
    keep = gt | ((tie > 0.0) & (prefix < need))
    out_ref[...] = jnp.where(keep, sg, 0.0)


def _fused_body(nv_ref, h_ref, sg_ref, wf_ref, bf_ref, out_ref):
    i = pl.program_id(1)
    nv = nv_ref[0]
    nvb = nv_ref[0, pl.ds(i * MBLK, MBLK), :]
    dflt0 = jax.lax.Precision.DEFAULT
    nvh, nvl = _split_hl(nv)
    nvbh, nvbl = _split_hl(nvb)
    a = _dot_t(nvbh, nvh, dflt0) + (_dot_t(nvbh, nvl, dflt0)
                                    + _dot_t(nvbl, nvh, dflt0))
    adj = jax.nn.relu(jnp.tanh(a))
    thr = _kth_threshold(adj, TOPK)
    dyn = jnp.where(adj >= thr, adj, 0.0)
    h = h_ref[0]
    dflt = jax.lax.Precision.DEFAULT
    hd = _dot(dyn, h, dflt)
    hs = _dot(sg_ref[pl.ds(i * MBLK, MBLK), :], h, dflt)
    hres = h_ref[0, pl.ds(i * MBLK, MBLK), :]
    fused = jax.nn.relu(hs + hd + hres)
    out_ref[0] = _dot(fused, wf_ref[...], dflt) + bf_ref[...]


def _pad_rows(arr, rows):
    return jnp.pad(arr, ((0, rows - arr.shape[0]),) + ((0, 0),) * (arr.ndim - 1))


@jax.jit
def kernel(history_data, TID, DIW, node_emb, node_emb_u, node_emb_d, emb1_w,
           emb2_w, Wts, bts, W1a, b1a, W1b, b1b, W1c, b1c, W2a, b2a, W2b,
           b2b, W2c, b2c, W_fore, b_fore):
    f32 = jnp.float32
    x = history_data.transpose(0, 2, 1, 3).reshape(B, N, F)
    x = jnp.pad(x, ((0, 0), (0, NPAD - N), (0, FPAD - F)))
    tid_idx = (history_data[:, -1, :, 1] * TOD).astype(jnp.int32)
    diw_idx = (history_data[:, -1, :, 2] * DOW).astype(jnp.int32)
    tid_idx = jnp.pad(tid_idx, ((0, 0), (0, NPAD - N)))[..., None]
    diw_idx = jnp.pad(diw_idx, ((0, 0), (0, NPAD - N)))[..., None]

    node_p = _pad_rows(node_emb, NPAD)
    emb1_p = _pad_rows(emb1_w, NPAD)
    eu_p = _pad_rows(node_emb_u, NPAD)
    ed_p = _pad_rows(node_emb_d, NPAD)
    diw_tab = _pad_rows(DIW, DOWP)
    wts_p = _pad_rows(Wts, FPAD)

    h, nv = pl.pallas_call(
        _prep_body,
        grid=(B,),
        in_specs=[
            pl.BlockSpec((1, NPAD, FPAD), lambda b: (b, 0, 0)),
            pl.BlockSpec((1, NPAD, 1), lambda b: (b, 0, 0)),
            pl.BlockSpec((1, NPAD, 1), lambda b: (b, 0, 0)),
            pl.BlockSpec((NPAD, EMBED), lambda b: (0, 0)),
            pl.BlockSpec((TOD, EMBED), lambda b: (0, 0)),
            pl.BlockSpec((DOWP, EMBED), lambda b: (0, 0)),
            pl.BlockSpec((FPAD, EMBED), lambda b: (0, 0)),
            pl.BlockSpec((1, EMBED), lambda b: (0, 0)),
            pl.BlockSpec((HID, 64), lambda b: (0, 0)),
            pl.BlockSpec((1, 64), lambda b: (0, 0)),
            pl.BlockSpec((64, 64), lambda b: (0, 0)),
            pl.BlockSpec((1, 64), lambda b: (0, 0)),
            pl.BlockSpec((64, NODE_DIM), lambda b: (0, 0)),
            pl.BlockSpec((1, NODE_DIM), lambda b: (0, 0)),
            pl.BlockSpec((NPAD, NODE_DIM), lambda b: (0, 0)),
        ],
        out_specs=[
            pl.BlockSpec((1, NPAD, HID), lambda b: (b, 0, 0)),
            pl.BlockSpec((1, NPAD, NODE_DIM), lambda b: (b, 0, 0)),
        ],
        out_shape=[
            jax.ShapeDtypeStruct((B, NPAD, HID), f32),
            jax.ShapeDtypeStruct((B, NPAD, NODE_DIM), f32),
        ],
    )(x, tid_idx, diw_idx, node_p, TID, diw_tab, wts_p, bts[None],
      W1a, b1a[None], W1b, b1b[None], W1c, b1c[None], emb1_p)

    static_graph = pl.pallas_call(
        _static_body,
        grid=(NBLK,),
        in_specs=[
            pl.BlockSpec((MBLK, EMBED), lambda i: (i, 0)),
            pl.BlockSpec((NPAD, EMBED), lambda i: (0, 0)),
        ],
        out_specs=pl.BlockSpec((MBLK, NPAD), lambda i: (i, 0)),
        out_shape=jax.ShapeDtypeStruct((NPAD, NPAD), f32),
    )(ed_p, eu_p)

    out = pl.pallas_call(
        _fused_body,
        grid=(B, NBLK),
        in_specs=[
            pl.BlockSpec((1, NPAD, NODE_DIM), lambda b, i: (b, 0, 0)),
            pl.BlockSpec((1, NPAD, HID), lambda b, i: (b, 0, 0)),
            pl.BlockSpec((NPAD, NPAD), lambda b, i: (0, 0)),
            pl.BlockSpec((HID, SEQ_OUT), lambda b, i: (0, 0)),
            pl.BlockSpec((1, SEQ_OUT), lambda b, i: (0, 0)),
        ],
        out_specs=pl.BlockSpec((1, MBLK, SEQ_OUT), lambda b, i: (b, i, 0)),
        out_shape=jax.ShapeDtypeStruct((B, NPAD, SEQ_OUT), f32),
    )(nv, h, static_graph, W_fore, b_fore[None])

    return out[:, :N, :]
